# baseline (speedup 1.0000x reference)
import functools

import jax
import jax.numpy as jnp
from jax import lax
from jax.experimental import pallas as pl
from jax.experimental.pallas import tpu as pltpu
from jax.experimental.pallas import tpu_sc as plsc

N = 10000
E = 320000
F = 128
EPS = 1e-5

NC = 2
NS = 16
NW = NC * NS
EPW = E // NW
K = 80
NCHUNK = EPW // K
NH = 5120
ND = 256
NR = NH + ND
RPS = NR // NS
NP = 2 * NH

_mesh = plsc.VectorSubcoreMesh(core_axis_name="c", subcore_axis_name="s")
_params = pltpu.CompilerParams(needs_layout_passes=False)


@functools.partial(
    pl.kernel,
    out_type=jax.ShapeDtypeStruct((NW, N), jnp.float32),
    mesh=_mesh,
    compiler_params=_params,
    scratch_types=[
        pltpu.VMEM((EPW,), jnp.int32),
        pltpu.VMEM((N,), jnp.float32),
    ],
)
def _deg_parts(dst_hbm, out_hbm, dst_v, deg_v):
    c = lax.axis_index("c")
    s = lax.axis_index("s")
    wid = s * NC + c

    zeros16 = jnp.zeros((16,), jnp.float32)

    def zero_body(i, carry):
        deg_v[pl.ds(i * 16, 16)] = zeros16
        return carry

    lax.fori_loop(0, N // 16, zero_body, 0)

    pltpu.sync_copy(dst_hbm.at[pl.ds(wid * EPW, EPW)], dst_v)

    ones16 = jnp.full((16,), 1.0, jnp.float32)

    def body(i, carry):
        idx = dst_v[pl.ds(i * 16, 16)]
        plsc.addupdate_scatter(deg_v, [idx], ones16)
        return carry

    lax.fori_loop(0, EPW // 16, body, 0)

    pltpu.sync_copy(deg_v, out_hbm.at[wid])


def _linear_body(x_ref, w_ref, parts_ref, g_ref):
    deg = jnp.sum(parts_ref[...], axis=1, keepdims=True) + 1.0
    dinv = lax.rsqrt(deg)
    h = jnp.dot(x_ref[...], w_ref[...], preferred_element_type=jnp.float32)
    g_ref[...] = h * dinv


_linear = pl.pallas_call(
    _linear_body,
    out_shape=jax.ShapeDtypeStruct((N, F), jnp.float32),
)


@functools.partial(
    pl.kernel,
    out_type=jax.ShapeDtypeStruct((NC, NP, F), jnp.float32),
    mesh=_mesh,
    compiler_params=_params,
    scratch_types=[
        pltpu.VMEM((EPW,), jnp.int32),
        pltpu.VMEM((EPW,), jnp.int32),
        pltpu.VMEM((K,), jnp.int32),
        pltpu.VMEM((K,), jnp.int32),
        pltpu.VMEM((K,), jnp.int32),
        pltpu.VMEM((K,), jnp.int32),
        pltpu.VMEM((K, F), jnp.float32),
        pltpu.VMEM((K, F), jnp.float32),
        pltpu.VMEM_SHARED((NR, F), jnp.float32),
        pltpu.VMEM_SHARED((NR, F), jnp.float32),
        pltpu.SemaphoreType.DMA,
        pltpu.SemaphoreType.DMA,
        pltpu.SemaphoreType.DMA,
        pltpu.SemaphoreType.DMA,
    ],
)
def _edge_scatter(src_hbm, dst_hbm, g_hbm, out_hbm,
                  sidx_v, didx_v, dlo_a, dhi_a, dlo_b, dhi_b, rows_a, rows_b,
                  acc_lo, acc_hi, sem_a, sem_b, sem_sa, sem_sb):
    c = lax.axis_index("c")
    s = lax.axis_index("s")
    wid = s * NC + c

    zeros16 = jnp.zeros((16,), jnp.float32)

    def zbuf_body(i, carry):
        r = i // (F // 16)
        j = i % (F // 16)
        rows_a[r, pl.ds(j * 16, 16)] = zeros16
        return carry

    lax.fori_loop(0, K * (F // 16), zbuf_body, 0)

    for k in range(4):
        pltpu.sync_copy(rows_a, acc_lo.at[pl.ds(s * RPS + k * K, K)])
        pltpu.sync_copy(rows_a, acc_hi.at[pl.ds(s * RPS + k * K, K)])
    pltpu.sync_copy(rows_a.at[pl.ds(0, RPS - 4 * K)],
                    acc_lo.at[pl.ds(s * RPS + 4 * K, RPS - 4 * K)])
    pltpu.sync_copy(rows_a.at[pl.ds(0, RPS - 4 * K)],
                    acc_hi.at[pl.ds(s * RPS + 4 * K, RPS - 4 * K)])

    pltpu.sync_copy(src_hbm.at[pl.ds(wid * EPW, EPW)], sidx_v)
    pltpu.sync_copy(dst_hbm.at[pl.ds(wid * EPW, EPW)], didx_v)

    plsc.subcore_barrier()

    def gather_start(i, rows_v, sem):
        pltpu.make_async_copy(
            g_hbm.at[sidx_v.at[pl.ds(i * K, K)]], rows_v, sem).start()

    def gather_wait(i, rows_v, sem):
        pltpu.make_async_copy(
            g_hbm.at[sidx_v.at[pl.ds(i * K, K)]], rows_v, sem).wait()

    def route(i, dlo_v, dhi_v):
        def route_body(gidx, carry2):
            d = didx_v[pl.ds(i * K + gidx * 16, 16)]
            dump = NH + (d & (ND - 1))
            in_lo = d < NH
            dlo_v[pl.ds(gidx * 16, 16)] = jnp.where(in_lo, d, dump)
            dhi_v[pl.ds(gidx * 16, 16)] = jnp.where(in_lo, dump, d - NH)
            return carry2

        lax.fori_loop(0, K // 16, route_body, 0)

    def scatter_start(rows_v, dlo_v, dhi_v, sem):
        pltpu.make_async_copy(rows_v, acc_lo.at[dlo_v], sem).start(add=True)
        pltpu.make_async_copy(rows_v, acc_hi.at[dhi_v], sem).start(add=True)

    def scatter_wait(rows_v, dlo_v, dhi_v, sem):
        pltpu.make_async_copy(rows_v, acc_lo.at[dlo_v], sem).wait()
        pltpu.make_async_copy(rows_v, acc_hi.at[dhi_v], sem).wait()

    gather_start(0, rows_a, sem_a)
    gather_wait(0, rows_a, sem_a)
    route(0, dlo_a, dhi_a)
    scatter_start(rows_a, dlo_a, dhi_a, sem_sa)
    gather_start(1, rows_b, sem_b)

    def body(j, carry):
        ib = 2 * j + 1
        gather_wait(ib, rows_b, sem_b)
        route(ib, dlo_b, dhi_b)
        scatter_wait(rows_a, dlo_a, dhi_a, sem_sa)
        scatter_start(rows_b, dlo_b, dhi_b, sem_sb)
        gather_start(ib + 1, rows_a, sem_a)
        gather_wait(ib + 1, rows_a, sem_a)
        route(ib + 1, dlo_a, dhi_a)
        scatter_wait(rows_b, dlo_b, dhi_b, sem_sb)
        scatter_start(rows_a, dlo_a, dhi_a, sem_sa)
        gather_start(ib + 2, rows_b, sem_b)
        return carry

    lax.fori_loop(0, (NCHUNK - 3) // 2, body, 0)

    ib = NCHUNK - 2
    gather_wait(ib, rows_b, sem_b)
    route(ib, dlo_b, dhi_b)
    scatter_wait(rows_a, dlo_a, dhi_a, sem_sa)
    scatter_start(rows_b, dlo_b, dhi_b, sem_sb)
    gather_start(ib + 1, rows_a, sem_a)
    gather_wait(ib + 1, rows_a, sem_a)
    route(ib + 1, dlo_a, dhi_a)
    scatter_wait(rows_b, dlo_b, dhi_b, sem_sb)
    scatter_start(rows_a, dlo_a, dhi_a, sem_sa)
    scatter_wait(rows_a, dlo_a, dhi_a, sem_sa)

    plsc.subcore_barrier()

    WB = NH // NS
    pltpu.sync_copy(acc_lo.at[pl.ds(s * WB, WB)],
                    out_hbm.at[c, pl.ds(s * WB, WB)])
    pltpu.sync_copy(acc_hi.at[pl.ds(s * WB, WB)],
                    out_hbm.at[c, pl.ds(NH + s * WB, WB)])


def _bn_body(accs_ref, g_ref, parts_ref, b_ref, gamma_ref, beta_ref,
             o_ref):
    deg = jnp.sum(parts_ref[...], axis=1, keepdims=True) + 1.0
    dinv = lax.rsqrt(deg)
    t = (accs_ref[0, 0:N, :] + accs_ref[1, 0:N, :] + g_ref[...]) * dinv
    t = t + b_ref[...]
    mu = jnp.mean(t, axis=0, keepdims=True)
    ms = jnp.mean(t * t, axis=0, keepdims=True)
    var = ms - mu * mu
    o_ref[...] = gamma_ref[...] * ((t - mu) * lax.rsqrt(var + EPS)) + beta_ref[...]


_bn = pl.pallas_call(
    _bn_body,
    out_shape=jax.ShapeDtypeStruct((N, F), jnp.float32),
)


def kernel(x, edge_index, W, b, gamma, beta):
    src = edge_index[0].astype(jnp.int32)
    dst = edge_index[1].astype(jnp.int32)
    parts = _deg_parts(dst)
    parts_t = parts.T
    g = _linear(x, W, parts_t)
    accs = _edge_scatter(src, dst, g)
    out = _bn(accs, g, parts_t,
              b.reshape(1, F), gamma.reshape(1, F), beta.reshape(1, F))
    return out


# speedup vs baseline: 1.0695x; 1.0695x over previous
import functools

import jax
import jax.numpy as jnp
from jax import lax
from jax.experimental import pallas as pl
from jax.experimental.pallas import tpu as pltpu
from jax.experimental.pallas import tpu_sc as plsc

N = 10000
E = 320000
F = 128
EPS = 1e-5

NC = 2
NS = 16
NW = NC * NS
EPW = E // NW
K = 80
NCHUNK = EPW // K
NH = 5120
ND = 256
NR = NH + ND
RPS = NR // NS
NP = 2 * NH

_mesh = plsc.VectorSubcoreMesh(core_axis_name="c", subcore_axis_name="s")
_params = pltpu.CompilerParams(needs_layout_passes=False)


@functools.partial(
    pl.kernel,
    out_type=jax.ShapeDtypeStruct((NW, N), jnp.float32),
    mesh=_mesh,
    compiler_params=_params,
    scratch_types=[
        pltpu.VMEM((EPW,), jnp.int32),
        pltpu.VMEM((N,), jnp.float32),
    ],
)
def _deg_parts(dst_hbm, out_hbm, dst_v, deg_v):
    c = lax.axis_index("c")
    s = lax.axis_index("s")
    wid = s * NC + c

    zeros16 = jnp.zeros((16,), jnp.float32)

    def zero_body(i, carry):
        deg_v[pl.ds(i * 16, 16)] = zeros16
        return carry

    lax.fori_loop(0, N // 16, zero_body, 0)

    pltpu.sync_copy(dst_hbm.at[pl.ds(wid * EPW, EPW)], dst_v)

    ones16 = jnp.full((16,), 1.0, jnp.float32)

    def body(i, carry):
        idx = dst_v[pl.ds(i * 16, 16)]
        plsc.addupdate_scatter(deg_v, [idx], ones16)
        return carry

    lax.fori_loop(0, EPW // 16, body, 0)

    pltpu.sync_copy(deg_v, out_hbm.at[wid])


def _linear_body(x_ref, w_ref, parts_ref, g_ref):
    deg = jnp.sum(parts_ref[...], axis=1, keepdims=True) + 1.0
    dinv = lax.rsqrt(deg)
    h = jnp.dot(x_ref[...], w_ref[...], preferred_element_type=jnp.float32)
    g_ref[...] = h * dinv


_linear = pl.pallas_call(
    _linear_body,
    out_shape=jax.ShapeDtypeStruct((N, F), jnp.float32),
)


@functools.partial(
    pl.kernel,
    out_type=jax.ShapeDtypeStruct((NC, NP, F), jnp.float32),
    mesh=_mesh,
    compiler_params=_params,
    scratch_types=[
        pltpu.VMEM((EPW,), jnp.int32),
        pltpu.VMEM((EPW,), jnp.int32),
        pltpu.VMEM((K,), jnp.int32),
        pltpu.VMEM((K,), jnp.int32),
        pltpu.VMEM((K,), jnp.int32),
        pltpu.VMEM((K,), jnp.int32),
        pltpu.VMEM((K, F), jnp.float32),
        pltpu.VMEM((K, F), jnp.float32),
        pltpu.VMEM_SHARED((NR, F), jnp.float32),
        pltpu.VMEM_SHARED((NR, F), jnp.float32),
        pltpu.SemaphoreType.DMA,
        pltpu.SemaphoreType.DMA,
        pltpu.SemaphoreType.DMA,
    ],
)
def _edge_scatter(src_hbm, dst_hbm, g_hbm, out_hbm,
                  sidx_v, didx_v, dlo_a, dhi_a, dlo_b, dhi_b, rows_a, rows_b,
                  acc_lo, acc_hi, sem_a, sem_b, sem_s):
    c = lax.axis_index("c")
    s = lax.axis_index("s")
    wid = s * NC + c

    zeros16 = jnp.zeros((16,), jnp.float32)

    def zbuf_body(i, carry):
        r = i // (F // 16)
        j = i % (F // 16)
        rows_a[r, pl.ds(j * 16, 16)] = zeros16
        return carry

    lax.fori_loop(0, K * (F // 16), zbuf_body, 0)

    for k in range(4):
        pltpu.sync_copy(rows_a, acc_lo.at[pl.ds(s * RPS + k * K, K)])
        pltpu.sync_copy(rows_a, acc_hi.at[pl.ds(s * RPS + k * K, K)])
    pltpu.sync_copy(rows_a.at[pl.ds(0, RPS - 4 * K)],
                    acc_lo.at[pl.ds(s * RPS + 4 * K, RPS - 4 * K)])
    pltpu.sync_copy(rows_a.at[pl.ds(0, RPS - 4 * K)],
                    acc_hi.at[pl.ds(s * RPS + 4 * K, RPS - 4 * K)])

    pltpu.sync_copy(src_hbm.at[pl.ds(wid * EPW, EPW)], sidx_v)
    pltpu.sync_copy(dst_hbm.at[pl.ds(wid * EPW, EPW)], didx_v)

    plsc.subcore_barrier()

    def gather_start(i, rows_v, sem):
        pltpu.make_async_copy(
            g_hbm.at[sidx_v.at[pl.ds(i * K, K)]], rows_v, sem).start()

    def route(i, dlo_v, dhi_v):
        def route_body(gidx, carry2):
            d = didx_v[pl.ds(i * K + gidx * 16, 16)]
            dump = NH + (d & (ND - 1))
            in_lo = d < NH
            dlo_v[pl.ds(gidx * 16, 16)] = jnp.where(in_lo, d, dump)
            dhi_v[pl.ds(gidx * 16, 16)] = jnp.where(in_lo, dump, d - NH)
            return carry2

        lax.fori_loop(0, K // 16, route_body, 0)

    def scatter(rows_v, dlo_v, dhi_v):
        d1 = pltpu.make_async_copy(rows_v, acc_lo.at[dlo_v], sem_s)
        d2 = pltpu.make_async_copy(rows_v, acc_hi.at[dhi_v], sem_s)
        d1.start(add=True)
        d2.start(add=True)
        d1.wait()
        d2.wait()

    gather_start(0, rows_a, sem_a)

    def body(j, carry):
        ia = 2 * j
        gather_start(ia + 1, rows_b, sem_b)
        pltpu.make_async_copy(g_hbm.at[sidx_v.at[pl.ds(ia * K, K)]],
                              rows_a, sem_a).wait()
        route(ia, dlo_a, dhi_a)
        scatter(rows_a, dlo_a, dhi_a)
        gather_start(ia + 2, rows_a, sem_a)
        pltpu.make_async_copy(g_hbm.at[sidx_v.at[pl.ds((ia + 1) * K, K)]],
                              rows_b, sem_b).wait()
        route(ia + 1, dlo_b, dhi_b)
        scatter(rows_b, dlo_b, dhi_b)
        return carry

    lax.fori_loop(0, (NCHUNK - 1) // 2, body, 0)

    last = NCHUNK - 1
    pltpu.make_async_copy(g_hbm.at[sidx_v.at[pl.ds(last * K, K)]],
                          rows_a, sem_a).wait()
    route(last, dlo_a, dhi_a)
    scatter(rows_a, dlo_a, dhi_a)

    plsc.subcore_barrier()

    WB = NH // NS
    pltpu.sync_copy(acc_lo.at[pl.ds(s * WB, WB)],
                    out_hbm.at[c, pl.ds(s * WB, WB)])
    pltpu.sync_copy(acc_hi.at[pl.ds(s * WB, WB)],
                    out_hbm.at[c, pl.ds(NH + s * WB, WB)])


def _bn_body(accs_ref, g_ref, parts_ref, b_ref, gamma_ref, beta_ref,
             o_ref):
    deg = jnp.sum(parts_ref[...], axis=1, keepdims=True) + 1.0
    dinv = lax.rsqrt(deg)
    t = (accs_ref[0, 0:N, :] + accs_ref[1, 0:N, :] + g_ref[...]) * dinv
    t = t + b_ref[...]
    mu = jnp.mean(t, axis=0, keepdims=True)
    ms = jnp.mean(t * t, axis=0, keepdims=True)
    var = ms - mu * mu
    o_ref[...] = gamma_ref[...] * ((t - mu) * lax.rsqrt(var + EPS)) + beta_ref[...]


_bn = pl.pallas_call(
    _bn_body,
    out_shape=jax.ShapeDtypeStruct((N, F), jnp.float32),
)


def kernel(x, edge_index, W, b, gamma, beta):
    src = edge_index[0].astype(jnp.int32)
    dst = edge_index[1].astype(jnp.int32)
    parts = _deg_parts(dst)
    parts_t = parts.T
    g = _linear(x, W, parts_t)
    accs = _edge_scatter(src, dst, g)
    out = _bn(accs, g, parts_t,
              b.reshape(1, F), gamma.reshape(1, F), beta.reshape(1, F))
    return out
